# baseline (speedup 1.0000x reference)
import functools

import jax
import jax.numpy as jnp
from jax import lax
from jax.experimental import pallas as pl
from jax.experimental.pallas import tpu as pltpu
from jax.experimental.pallas import tpu_sc as plsc

N = 10000
E = 320000
D = 128

NC = 2
NS = 16
NW = NC * NS
CH = 128
NCHG = E // CH
NCH = NCHG // NW
NEXTRA = NCHG - NCH * NW
RING = 3

RPT = 632
RPT_LAST = N - (NS - 1) * RPT

_mesh = plsc.VectorSubcoreMesh(core_axis_name="c", subcore_axis_name="s")


@functools.partial(
    pl.kernel,
    out_type=jax.ShapeDtypeStruct((NC, N), jnp.float32),
    mesh=_mesh,
    scratch_types=[
        pltpu.VMEM((NCH + 1, CH), jnp.int32),
        pltpu.VMEM((CH,), jnp.float32),
        pltpu.VMEM_SHARED((N,), jnp.float32),
        pltpu.SemaphoreType.DMA,
        pltpu.SemaphoreType.DMA,
    ],
)
def _deg_kernel(ei_hbm, zeros1_hbm, out_hbm, idx_v, ones_v, deg_sh,
                seml, sems):
    cid = lax.axis_index("c")
    sid = lax.axis_index("s")
    wid = sid * NC + cid
    c0 = wid * NCH

    @pl.when(sid == 0)
    def _():
        pltpu.sync_copy(zeros1_hbm, deg_sh)

    def fire_load(j, _):
        pltpu.async_copy(ei_hbm.at[1, pl.ds((c0 + j) * CH, CH)],
                         idx_v.at[j], seml)
        return 0

    lax.fori_loop(0, NCH, fire_load, 0)

    @pl.when(wid < NEXTRA)
    def _():
        pltpu.async_copy(ei_hbm.at[1, pl.ds((NCH * NW + wid) * CH, CH)],
                         idx_v.at[NCH], seml)

    for i in range(CH // 16):
        ones_v[pl.ds(i * 16, 16)] = jnp.ones((16,), jnp.float32)

    def drain_load(j, _):
        pltpu.make_async_copy(ei_hbm.at[1, pl.ds((c0 + j) * CH, CH)],
                              idx_v.at[j], seml).wait()
        return 0

    lax.fori_loop(0, NCH, drain_load, 0)

    @pl.when(wid < NEXTRA)
    def _():
        pltpu.make_async_copy(ei_hbm.at[1, pl.ds((NCH * NW + wid) * CH, CH)],
                              idx_v.at[NCH], seml).wait()

    plsc.subcore_barrier()

    def fire_scatter(j, _):
        pltpu.async_copy(ones_v, deg_sh.at[idx_v.at[j]], sems, add=True)
        return 0

    lax.fori_loop(0, NCH, fire_scatter, 0)

    @pl.when(wid < NEXTRA)
    def _():
        pltpu.async_copy(ones_v, deg_sh.at[idx_v.at[NCH]], sems, add=True)

    def drain_scatter(j, _):
        pltpu.make_async_copy(ones_v, deg_sh.at[idx_v.at[j]], sems).wait()
        return 0

    lax.fori_loop(0, NCH, drain_scatter, 0)

    @pl.when(wid < NEXTRA)
    def _():
        pltpu.make_async_copy(ones_v, deg_sh.at[idx_v.at[NCH]], sems).wait()

    plsc.subcore_barrier()

    @pl.when(sid == 0)
    def _():
        pltpu.sync_copy(deg_sh, out_hbm.at[cid])


@functools.partial(
    pl.kernel,
    out_type=jax.ShapeDtypeStruct((NC, N, D), jnp.float32),
    mesh=_mesh,
    scratch_types=[
        pltpu.VMEM((CH,), jnp.int32),
        pltpu.VMEM((CH,), jnp.int32),
        pltpu.VMEM((CH,), jnp.int32),
        pltpu.VMEM((CH,), jnp.int32),
        pltpu.VMEM((CH,), jnp.int32),
        pltpu.VMEM((CH,), jnp.int32),
        pltpu.VMEM((CH, D), jnp.float32),
        pltpu.VMEM((CH, D), jnp.float32),
        pltpu.VMEM((CH, D), jnp.float32),
        pltpu.VMEM_SHARED((N, D), jnp.float32),
        pltpu.SemaphoreType.DMA,
        pltpu.SemaphoreType.DMA,
        pltpu.SemaphoreType.DMA,
        pltpu.SemaphoreType.DMA,
        pltpu.SemaphoreType.DMA,
        pltpu.SemaphoreType.DMA,
    ],
)
def _agg_kernel(ei_hbm, y_hbm, out_hbm,
                sb0, sb1, sb2, db0, db1, db2,
                rows0_v, rows1_v, rows2_v, acc_sh,
                semg0, semg1, semg2, semi0, semi1, semi2):
    cid = lax.axis_index("c")
    sid = lax.axis_index("s")
    wid = sid * NC + cid
    r0 = sid * RPT
    c0 = wid * NCH
    sb = [sb0, sb1, sb2]
    db = [db0, db1, db2]
    rows = [rows0_v, rows1_v, rows2_v]
    semg = [semg0, semg1, semg2]
    semi = [semi0, semi1, semi2]

    def load_idx(c, b):
        pltpu.async_copy(ei_hbm.at[0, pl.ds(c * CH, CH)], sb[b], semi[b])
        pltpu.async_copy(ei_hbm.at[1, pl.ds(c * CH, CH)], db[b], semi[b])

    def wait_idx(c, b):
        pltpu.make_async_copy(ei_hbm.at[0, pl.ds(c * CH, CH)], sb[b],
                              semi[b]).wait()
        pltpu.make_async_copy(ei_hbm.at[1, pl.ds(c * CH, CH)], db[b],
                              semi[b]).wait()

    for b in range(RING):
        load_idx(c0 + b, b)

    def zrow(r, _):
        for c in range(D // 16):
            rows0_v[r, pl.ds(c * 16, 16)] = jnp.zeros((16,), jnp.float32)
        return 0

    lax.fori_loop(0, CH, zrow, 0)
    for k in range(4):
        pltpu.sync_copy(rows0_v, acc_sh.at[pl.ds(r0 + k * CH, CH)])

    @pl.when(sid < NS - 1)
    def _():
        pltpu.sync_copy(rows0_v.at[pl.ds(0, RPT - 4 * CH)],
                        acc_sh.at[pl.ds(r0 + 4 * CH, RPT - 4 * CH)])

    @pl.when(sid == NS - 1)
    def _():
        pltpu.sync_copy(rows0_v.at[pl.ds(0, RPT_LAST - 4 * CH)],
                        acc_sh.at[pl.ds(r0 + 4 * CH, RPT_LAST - 4 * CH)])

    plsc.subcore_barrier()

    for b in range(2):
        wait_idx(c0 + b, b)
        pltpu.async_copy(y_hbm.at[sb[b]], rows[b], semg[b])

    def outer(g, _):
        for b in range(RING):
            j = g * RING + b
            nb = (b + 2) % RING

            @pl.when(j + 2 < NCH)
            def _(j=j, nb=nb):
                wait_idx(c0 + j + 2, nb)
                pltpu.async_copy(y_hbm.at[sb[nb]], rows[nb], semg[nb])

            pltpu.make_async_copy(y_hbm.at[sb[b]], rows[b], semg[b]).wait()
            pltpu.sync_copy(rows[b], acc_sh.at[db[b]], add=True)

            @pl.when(j + 3 < NCH)
            def _(j=j, b=b):
                load_idx(c0 + j + 3, b)
        return 0

    lax.fori_loop(0, NCH // RING, outer, 0)

    @pl.when(wid < NEXTRA)
    def _():
        ce = NCH * NW + wid
        load_idx(ce, 0)
        wait_idx(ce, 0)
        pltpu.async_copy(y_hbm.at[sb[0]], rows[0], semg[0])
        pltpu.make_async_copy(y_hbm.at[sb[0]], rows[0], semg[0]).wait()
        pltpu.sync_copy(rows[0], acc_sh.at[db[0]], add=True)

    plsc.subcore_barrier()

    @pl.when(sid < NS - 1)
    def _():
        pltpu.sync_copy(acc_sh.at[pl.ds(r0, RPT)],
                        out_hbm.at[cid, pl.ds(r0, RPT)])

    @pl.when(sid == NS - 1)
    def _():
        pltpu.sync_copy(acc_sh.at[pl.ds(r0, RPT_LAST)],
                        out_hbm.at[cid, pl.ds(r0, RPT_LAST)])


BR = 2000


def _y_body(deg_ref, x_ref, w_ref, y_ref):
    d = deg_ref[0, :, 0] + deg_ref[1, :, 0] + 1.0
    dinv = lax.rsqrt(d)
    y_ref[...] = jnp.dot(x_ref[...] * dinv[:, None], w_ref[...],
                         preferred_element_type=jnp.float32)


_y_call = pl.pallas_call(
    _y_body,
    grid=(N // BR,),
    in_specs=[
        pl.BlockSpec((NC, BR, 1), lambda i: (0, i, 0)),
        pl.BlockSpec((BR, D), lambda i: (i, 0)),
        pl.BlockSpec((D, D), lambda i: (0, 0)),
    ],
    out_specs=pl.BlockSpec((BR, D), lambda i: (i, 0)),
    out_shape=jax.ShapeDtypeStruct((N, D), jnp.float32),
)


BR4 = 400


def _out_body(deg_ref, acc_ref, y_ref, b_ref, o_ref):
    d = deg_ref[0, :, 0] + deg_ref[1, :, 0] + 1.0
    dinv = lax.rsqrt(d)
    o_ref[...] = ((acc_ref[0] + acc_ref[1] + y_ref[...]) * dinv[:, None]
                  + b_ref[...])


_out_call = pl.pallas_call(
    _out_body,
    grid=(N // BR4,),
    in_specs=[
        pl.BlockSpec((NC, BR4, 1), lambda i: (0, i, 0)),
        pl.BlockSpec((NC, BR4, D), lambda i: (0, i, 0)),
        pl.BlockSpec((BR4, D), lambda i: (i, 0)),
        pl.BlockSpec((1, D), lambda i: (0, 0)),
    ],
    out_specs=pl.BlockSpec((BR4, D), lambda i: (i, 0)),
    out_shape=jax.ShapeDtypeStruct((N, D), jnp.float32),
)


def kernel(x, edge_index, W, b):
    ei = edge_index.astype(jnp.int32)
    zeros1 = jnp.zeros((N,), jnp.float32)

    degp = _deg_kernel(ei, zeros1)
    degp3 = degp.reshape(NC, N, 1)
    y = _y_call(degp3, x, W)
    accp = _agg_kernel(ei, y)
    return _out_call(degp3, accp, y, b.reshape(1, D))


# speedup vs baseline: 1.0561x; 1.0561x over previous
import functools

import jax
import jax.numpy as jnp
from jax import lax
from jax.experimental import pallas as pl
from jax.experimental.pallas import tpu as pltpu
from jax.experimental.pallas import tpu_sc as plsc

N = 10000
E = 320000
D = 128

NC = 2
NS = 16
NW = NC * NS
CH = 128
NCHG = E // CH
NCH = NCHG // NW
NEXTRA = NCHG - NCH * NW
RING = 3

RPT = 632
RPT_LAST = N - (NS - 1) * RPT

_mesh = plsc.VectorSubcoreMesh(core_axis_name="c", subcore_axis_name="s")


@functools.partial(
    pl.kernel,
    out_type=jax.ShapeDtypeStruct((NC, N), jnp.float32),
    mesh=_mesh,
    scratch_types=[
        pltpu.VMEM((NCH + 1, CH), jnp.int32),
        pltpu.VMEM((CH,), jnp.float32),
        pltpu.VMEM_SHARED((N,), jnp.float32),
        pltpu.SemaphoreType.DMA,
        pltpu.SemaphoreType.DMA,
    ],
)
def _deg_kernel(ei_hbm, zeros1_hbm, out_hbm, idx_v, ones_v, deg_sh,
                seml, sems):
    cid = lax.axis_index("c")
    sid = lax.axis_index("s")
    wid = sid * NC + cid
    c0 = wid * NCH

    @pl.when(sid == 0)
    def _():
        pltpu.sync_copy(zeros1_hbm, deg_sh)

    def fire_load(j, _):
        pltpu.async_copy(ei_hbm.at[1, pl.ds((c0 + j) * CH, CH)],
                         idx_v.at[j], seml)
        return 0

    lax.fori_loop(0, NCH, fire_load, 0)

    @pl.when(wid < NEXTRA)
    def _():
        pltpu.async_copy(ei_hbm.at[1, pl.ds((NCH * NW + wid) * CH, CH)],
                         idx_v.at[NCH], seml)

    for i in range(CH // 16):
        ones_v[pl.ds(i * 16, 16)] = jnp.ones((16,), jnp.float32)

    def drain_load(j, _):
        pltpu.make_async_copy(ei_hbm.at[1, pl.ds((c0 + j) * CH, CH)],
                              idx_v.at[j], seml).wait()
        return 0

    lax.fori_loop(0, NCH, drain_load, 0)

    @pl.when(wid < NEXTRA)
    def _():
        pltpu.make_async_copy(ei_hbm.at[1, pl.ds((NCH * NW + wid) * CH, CH)],
                              idx_v.at[NCH], seml).wait()

    plsc.subcore_barrier()

    def fire_scatter(j, _):
        pltpu.async_copy(ones_v, deg_sh.at[idx_v.at[j]], sems, add=True)
        return 0

    lax.fori_loop(0, NCH, fire_scatter, 0)

    @pl.when(wid < NEXTRA)
    def _():
        pltpu.async_copy(ones_v, deg_sh.at[idx_v.at[NCH]], sems, add=True)

    def drain_scatter(j, _):
        pltpu.make_async_copy(ones_v, deg_sh.at[idx_v.at[j]], sems).wait()
        return 0

    lax.fori_loop(0, NCH, drain_scatter, 0)

    @pl.when(wid < NEXTRA)
    def _():
        pltpu.make_async_copy(ones_v, deg_sh.at[idx_v.at[NCH]], sems).wait()

    plsc.subcore_barrier()

    @pl.when(sid == 0)
    def _():
        pltpu.sync_copy(deg_sh, out_hbm.at[cid])


@functools.partial(
    pl.kernel,
    out_type=jax.ShapeDtypeStruct((NC, N, D), jnp.float32),
    mesh=_mesh,
    scratch_types=[
        pltpu.VMEM((CH,), jnp.int32),
        pltpu.VMEM((CH,), jnp.int32),
        pltpu.VMEM((CH,), jnp.int32),
        pltpu.VMEM((CH,), jnp.int32),
        pltpu.VMEM((CH,), jnp.int32),
        pltpu.VMEM((CH,), jnp.int32),
        pltpu.VMEM((CH, D), jnp.float32),
        pltpu.VMEM((CH, D), jnp.float32),
        pltpu.VMEM((CH, D), jnp.float32),
        pltpu.VMEM_SHARED((N, D), jnp.float32),
        pltpu.SemaphoreType.DMA,
        pltpu.SemaphoreType.DMA,
        pltpu.SemaphoreType.DMA,
        pltpu.SemaphoreType.DMA,
        pltpu.SemaphoreType.DMA,
        pltpu.SemaphoreType.DMA,
    ],
)
def _agg_kernel(ei_hbm, y_hbm, out_hbm,
                sb0, sb1, sb2, db0, db1, db2,
                rows0_v, rows1_v, rows2_v, acc_sh,
                semg0, semg1, semg2, semi0, semi1, semi2):
    cid = lax.axis_index("c")
    sid = lax.axis_index("s")
    wid = sid * NC + cid
    r0 = sid * RPT
    c0 = wid * NCH
    sb = [sb0, sb1, sb2]
    db = [db0, db1, db2]
    rows = [rows0_v, rows1_v, rows2_v]
    semg = [semg0, semg1, semg2]
    semi = [semi0, semi1, semi2]

    def load_idx(c, b):
        pltpu.async_copy(ei_hbm.at[0, pl.ds(c * CH, CH)], sb[b], semi[b])
        pltpu.async_copy(ei_hbm.at[1, pl.ds(c * CH, CH)], db[b], semi[b])

    def wait_idx(c, b):
        pltpu.make_async_copy(ei_hbm.at[0, pl.ds(c * CH, CH)], sb[b],
                              semi[b]).wait()
        pltpu.make_async_copy(ei_hbm.at[1, pl.ds(c * CH, CH)], db[b],
                              semi[b]).wait()

    for b in range(RING):
        load_idx(c0 + b, b)

    def zrow(r, _):
        for c in range(D // 16):
            rows0_v[r, pl.ds(c * 16, 16)] = jnp.zeros((16,), jnp.float32)
        return 0

    lax.fori_loop(0, CH, zrow, 0)
    for k in range(4):
        pltpu.sync_copy(rows0_v, acc_sh.at[pl.ds(r0 + k * CH, CH)])

    @pl.when(sid < NS - 1)
    def _():
        pltpu.sync_copy(rows0_v.at[pl.ds(0, RPT - 4 * CH)],
                        acc_sh.at[pl.ds(r0 + 4 * CH, RPT - 4 * CH)])

    @pl.when(sid == NS - 1)
    def _():
        pltpu.sync_copy(rows0_v.at[pl.ds(0, RPT_LAST - 4 * CH)],
                        acc_sh.at[pl.ds(r0 + 4 * CH, RPT_LAST - 4 * CH)])

    plsc.subcore_barrier()

    for b in range(2):
        wait_idx(c0 + b, b)
        pltpu.async_copy(y_hbm.at[sb[b]], rows[b], semg[b])

    def outer(g, _):
        for b in range(RING):
            j = g * RING + b
            nb = (b + 2) % RING

            @pl.when(j + 2 < NCH)
            def _(j=j, nb=nb):
                wait_idx(c0 + j + 2, nb)
                pltpu.async_copy(y_hbm.at[sb[nb]], rows[nb], semg[nb])

            pltpu.make_async_copy(y_hbm.at[sb[b]], rows[b], semg[b]).wait()
            pltpu.sync_copy(rows[b], acc_sh.at[db[b]], add=True)

            @pl.when(j + 3 < NCH)
            def _(j=j, b=b):
                load_idx(c0 + j + 3, b)
        return 0

    lax.fori_loop(0, NCH // RING, outer, 0)

    @pl.when(wid < NEXTRA)
    def _():
        ce = NCH * NW + wid
        load_idx(ce, 0)
        wait_idx(ce, 0)
        pltpu.async_copy(y_hbm.at[sb[0]], rows[0], semg[0])
        pltpu.make_async_copy(y_hbm.at[sb[0]], rows[0], semg[0]).wait()
        pltpu.sync_copy(rows[0], acc_sh.at[db[0]], add=True)

    plsc.subcore_barrier()

    @pl.when(sid < NS - 1)
    def _():
        pltpu.sync_copy(acc_sh.at[pl.ds(r0, RPT)],
                        out_hbm.at[cid, pl.ds(r0, RPT)])

    @pl.when(sid == NS - 1)
    def _():
        pltpu.sync_copy(acc_sh.at[pl.ds(r0, RPT_LAST)],
                        out_hbm.at[cid, pl.ds(r0, RPT_LAST)])


BR = 2000


def _y_body(deg_ref, x_ref, w_ref, y_ref):
    d = deg_ref[0, :, 0] + deg_ref[1, :, 0] + 1.0
    dinv = lax.rsqrt(d)
    y_ref[...] = jnp.dot(x_ref[...] * dinv[:, None], w_ref[...],
                         preferred_element_type=jnp.float32)


_y_call = pl.pallas_call(
    _y_body,
    grid=(N // BR,),
    in_specs=[
        pl.BlockSpec((NC, BR, 1), lambda i: (0, i, 0)),
        pl.BlockSpec((BR, D), lambda i: (i, 0)),
        pl.BlockSpec((D, D), lambda i: (0, 0)),
    ],
    out_specs=pl.BlockSpec((BR, D), lambda i: (i, 0)),
    out_shape=jax.ShapeDtypeStruct((N, D), jnp.float32),
)


def _out_body(deg_ref, acc_ref, y_ref, b_ref, o_ref):
    d = deg_ref[0, :, 0] + deg_ref[1, :, 0] + 1.0
    dinv = lax.rsqrt(d)
    o_ref[...] = ((acc_ref[0] + acc_ref[1] + y_ref[...]) * dinv[:, None]
                  + b_ref[...])


_out_call = pl.pallas_call(
    _out_body,
    grid=(N // BR,),
    in_specs=[
        pl.BlockSpec((NC, BR, 1), lambda i: (0, i, 0)),
        pl.BlockSpec((NC, BR, D), lambda i: (0, i, 0)),
        pl.BlockSpec((BR, D), lambda i: (i, 0)),
        pl.BlockSpec((1, D), lambda i: (0, 0)),
    ],
    out_specs=pl.BlockSpec((BR, D), lambda i: (i, 0)),
    out_shape=jax.ShapeDtypeStruct((N, D), jnp.float32),
)


def kernel(x, edge_index, W, b):
    ei = edge_index.astype(jnp.int32)
    zeros1 = jnp.zeros((N,), jnp.float32)

    degp = _deg_kernel(ei, zeros1)
    degp3 = degp.reshape(NC, N, 1)
    y = _y_call(degp3, x, W)
    accp = _agg_kernel(ei, y)
    return _out_call(degp3, accp, y, b.reshape(1, D))
